# baseline (speedup 1.0000x reference)
import functools

import jax
import jax.numpy as jnp
from jax import lax
from jax.experimental import pallas as pl
from jax.experimental.pallas import tpu as pltpu
from jax.experimental.pallas import tpu_sc as plsc

B = 4096
F = 26
D_DENSE = 13
FEAT = 4000
DIM = 16
TOTAL = F * FEAT
WCOL = F * DIM
ROW = 512

NC = 2
NS = 16
NW = NC * NS
NB = B // NW
CHUNK = 4
NCHUNK = NB // CHUNK
ROWS_PER_CHUNK = CHUNK * F

BT = 1000


def _pad_weights_body(w_ref, o_ref):
    w = w_ref[0, 0, :].reshape(BT, 1)
    o_ref[...] = jnp.concatenate(
        [w, jnp.zeros((BT, DIM - 1), jnp.float32)], axis=1)


def _pad_weights(weight_sparse):
    return pl.pallas_call(
        _pad_weights_body,
        grid=(TOTAL // BT,),
        in_specs=[pl.BlockSpec((1, 1, BT), lambda t: (t, 0, 0))],
        out_specs=pl.BlockSpec((BT, DIM), lambda t: (t, 0)),
        out_shape=jax.ShapeDtypeStruct((TOTAL, DIM), jnp.float32),
    )(weight_sparse.reshape(TOTAL // BT, 1, BT))


def _sc_gather_cross(emb, w16, sp_flat):
    mesh = plsc.VectorSubcoreMesh(core_axis_name="c", subcore_axis_name="s")

    @functools.partial(
        pl.kernel,
        mesh=mesh,
        out_type=jax.ShapeDtypeStruct((B, DIM), jnp.float32),
        scratch_types=[
            pltpu.VMEM((NB * F,), jnp.int32),
            pltpu.VMEM(((F + 1) * ROWS_PER_CHUNK, DIM), jnp.float32),
            pltpu.VMEM((NB, DIM), jnp.float32),
            pltpu.SemaphoreType.DMA,
        ],
        compiler_params=pltpu.CompilerParams(use_tc_tiling_on_sc=False),
    )
    def k(emb_hbm, w16_hbm, sp_hbm, out_hbm, idx_v, rows_v, out_v, sem):
        wid = lax.axis_index("s") * NC + lax.axis_index("c")
        base = wid * (NB * F)
        pltpu.sync_copy(sp_hbm.at[pl.ds(base, NB * F)], idx_v)

        def chunk_body(c, carry):
            ids = idx_v.at[pl.ds(c * ROWS_PER_CHUNK, ROWS_PER_CHUNK)]
            copies = [pltpu.async_copy(
                emb_hbm.at[j].at[ids],
                rows_v.at[pl.ds(j * ROWS_PER_CHUNK, ROWS_PER_CHUNK)],
                sem) for j in range(F)]
            copies.append(pltpu.async_copy(
                w16_hbm.at[ids],
                rows_v.at[pl.ds(F * ROWS_PER_CHUNK, ROWS_PER_CHUNK)], sem))
            for cp in copies:
                cp.wait()

            def b_body(bb, carry2):
                r0 = bb * F
                acc = jnp.zeros((DIM,), jnp.float32)
                for i in range(F - 1):
                    for j in range(i + 1, F):
                        acc = acc + (
                            rows_v[j * ROWS_PER_CHUNK + r0 + i, :] *
                            rows_v[i * ROWS_PER_CHUNK + r0 + j, :])
                for i in range(F):
                    acc = acc + rows_v[F * ROWS_PER_CHUNK + r0 + i, :]
                out_v[c * CHUNK + bb, :] = acc
                return carry2

            lax.fori_loop(0, CHUNK, b_body, 0, unroll=False)
            return carry

        lax.fori_loop(0, NCHUNK, chunk_body, 0, unroll=False)
        pltpu.sync_copy(out_v, out_hbm.at[pl.ds(wid * NB, NB)])

    return k(emb, w16, sp_flat)


def _final_body(dense_ref, wd_ref, b_ref, part_ref, o_ref):
    lin = jnp.sum(dense_ref[...] * wd_ref[...], axis=1, keepdims=True)
    cross = jnp.sum(part_ref[...], axis=1, keepdims=True)
    o_ref[...] = jax.nn.sigmoid(lin + cross + b_ref[0, 0])


def _final(dense, wd_row, bias11, partial):
    return pl.pallas_call(
        _final_body,
        out_shape=jax.ShapeDtypeStruct((B, 1), jnp.float32),
    )(dense, wd_row, bias11, partial)


def kernel(dense_input, sparse_input, bias, weight_dense, weight_sparse,
           embed_tables):
    offs = jnp.arange(F, dtype=jnp.int32) * FEAT
    sp_flat = (sparse_input + offs[None, :]).reshape(B * F)
    w16 = _pad_weights(weight_sparse)
    partial = _sc_gather_cross(embed_tables, w16, sp_flat)
    return _final(dense_input, weight_dense.reshape(1, D_DENSE),
                  bias.reshape(1, 1), partial)


# speedup vs baseline: 1.1866x; 1.1866x over previous
import functools

import jax
import jax.numpy as jnp
from jax import lax
from jax.experimental import pallas as pl
from jax.experimental.pallas import tpu as pltpu
from jax.experimental.pallas import tpu_sc as plsc

B = 4096
F = 26
D_DENSE = 13
FEAT = 4000
DIM = 16
TOTAL = F * FEAT
WCOL = F * DIM
ROW = 512

NC = 2
NS = 16
NW = NC * NS
NB = B // NW
CHUNK = 4
NCHUNK = NB // CHUNK
ROWS_PER_CHUNK = CHUNK * F


def _sc_gather_cross(table, sp_flat):
    mesh = plsc.VectorSubcoreMesh(core_axis_name="c", subcore_axis_name="s")

    @functools.partial(
        pl.kernel,
        mesh=mesh,
        out_type=jax.ShapeDtypeStruct((B, DIM), jnp.float32),
        scratch_types=[
            pltpu.VMEM((NB * F,), jnp.int32),
            pltpu.VMEM((ROWS_PER_CHUNK, ROW), jnp.float32),
            pltpu.VMEM((NB, DIM), jnp.float32),
            pltpu.SemaphoreType.DMA,
        ],
    )
    def k(table_hbm, sp_hbm, out_hbm, idx_v, rows_v, out_v, sem):
        wid = lax.axis_index("s") * NC + lax.axis_index("c")
        base = wid * (NB * F)
        pltpu.sync_copy(sp_hbm.at[pl.ds(base, NB * F)], idx_v)

        def chunk_body(c, carry):
            pltpu.async_copy(
                table_hbm.at[idx_v.at[pl.ds(c * ROWS_PER_CHUNK,
                                            ROWS_PER_CHUNK)]],
                rows_v, sem).wait()

            def b_body(bb, carry2):
                r0 = bb * F
                acc = jnp.zeros((DIM,), jnp.float32)
                for i in range(F - 1):
                    for j in range(i + 1, F):
                        acc = acc + (rows_v[r0 + i, pl.ds(j * DIM, DIM)] *
                                     rows_v[r0 + j, pl.ds(i * DIM, DIM)])
                for i in range(F):
                    acc = acc + rows_v[r0 + i, pl.ds(WCOL, DIM)]
                out_v[c * CHUNK + bb, :] = acc
                return carry2

            lax.fori_loop(0, CHUNK, b_body, 0, unroll=False)
            return carry

        lax.fori_loop(0, NCHUNK, chunk_body, 0, unroll=False)
        pltpu.sync_copy(out_v, out_hbm.at[pl.ds(wid * NB, NB)])

    return k(table, sp_flat)


def _final_body(dense_ref, wd_ref, b_ref, part_ref, o_ref):
    lin = jnp.sum(dense_ref[...] * wd_ref[...], axis=1, keepdims=True)
    cross = jnp.sum(part_ref[...], axis=1, keepdims=True)
    o_ref[...] = jax.nn.sigmoid(lin + cross + b_ref[0, 0])


def _final(dense, wd_row, bias11, partial):
    return pl.pallas_call(
        _final_body,
        out_shape=jax.ShapeDtypeStruct((B, 1), jnp.float32),
    )(dense, wd_row, bias11, partial)


def kernel(dense_input, sparse_input, bias, weight_dense, weight_sparse,
           embed_tables):
    offs = jnp.arange(F, dtype=jnp.int32) * FEAT
    sp_flat = (sparse_input + offs[None, :]).reshape(B * F)
    table = jnp.concatenate(
        [jnp.transpose(embed_tables, (1, 0, 2)).reshape(TOTAL, WCOL),
         weight_sparse,
         jnp.zeros((TOTAL, ROW - WCOL - 1), jnp.float32)], axis=1)
    partial = _sc_gather_cross(table, sp_flat)
    return _final(dense_input, weight_dense.reshape(1, D_DENSE),
                  bias.reshape(1, 1), partial)
